# baseline (speedup 1.0000x reference)
import functools

import jax
import jax.numpy as jnp
from jax import lax
from jax.experimental import pallas as pl
from jax.experimental.pallas import tpu as pltpu
from jax.experimental.pallas import tpu_sc as plsc

_GROUPS = ((0, 1), (1, 4), (4, 10))
_R, _A, _C = 4, 10, 8
_D = _R * _A * _C
_L = 16
_NW = 32
_BLK = 64
_SMAX = 5056


def _expand_params(p):
    parts = [jnp.broadcast_to(p[g][:, None, :], (_R, e - s, _C))
             for g, (s, e) in enumerate(_GROUPS)]
    return jnp.concatenate(parts, axis=1).reshape(_D)


def _make_sc_call(E, N):
    nblk = E // _BLK
    nk_hi = -(-nblk // _NW)
    nk_lo = nblk // _NW
    mesh = plsc.VectorSubcoreMesh(core_axis_name="c", subcore_axis_name="s")
    njc = _D // _L

    @functools.partial(
        pl.kernel,
        mesh=mesh,
        compiler_params=pltpu.CompilerParams(use_tc_tiling_on_sc=False),
        out_type=jax.ShapeDtypeStruct((E * _D,), jnp.float32),
        scratch_types=(
            [pltpu.VMEM((_SMAX,), jnp.int32),
             pltpu.VMEM((_SMAX,), jnp.float32),
             pltpu.VMEM((_SMAX,), jnp.float32)]
            + [pltpu.VMEM((_BLK, _D), jnp.float32) for _ in range(2)]
            + [pltpu.VMEM((_BLK * _D,), jnp.float32) for _ in range(2)]
            + [pltpu.VMEM((_D,), jnp.float32), pltpu.VMEM((_D,), jnp.float32)]
            + [pltpu.SemaphoreType.DMA for _ in range(4)]
        ),
    )
    def sc_kernel(table, src, el, cf, inv, pre, out,
                  sidx, sel, scf, rows0, rows1, ob0, ob1,
                  inv_v, pre_v, gsem0, gsem1, osem0, osem1):
        rows = (rows0, rows1)
        obuf = (ob0, ob1)
        gsem = (gsem0, gsem1)
        osem = (osem0, osem1)

        wid = lax.axis_index("s") * 2 + lax.axis_index("c")
        s_w = (wid * nblk) // _NW
        s_n = ((wid + 1) * nblk) // _NW
        nk = s_n - s_w
        ebase_w = s_w * _BLK

        pltpu.sync_copy(inv, inv_v)
        pltpu.sync_copy(pre, pre_v)
        inv_vecs = [inv_v[pl.ds(j * _L, _L)] for j in range(njc)]
        pre_vecs = [pre_v[pl.ds(j * _L, _L)] for j in range(njc)]

        @pl.when(nk == nk_hi)
        def _():
            pltpu.sync_copy(src.at[pl.ds(ebase_w, nk_hi * _BLK)],
                            sidx.at[pl.ds(0, nk_hi * _BLK)])
            pltpu.sync_copy(el.at[pl.ds(ebase_w, nk_hi * _BLK)],
                            sel.at[pl.ds(0, nk_hi * _BLK)])
            pltpu.sync_copy(cf.at[pl.ds(ebase_w, nk_hi * _BLK)],
                            scf.at[pl.ds(0, nk_hi * _BLK)])

        @pl.when(nk == nk_lo)
        def _():
            pltpu.sync_copy(src.at[pl.ds(ebase_w, nk_lo * _BLK)],
                            sidx.at[pl.ds(0, nk_lo * _BLK)])
            pltpu.sync_copy(el.at[pl.ds(ebase_w, nk_lo * _BLK)],
                            sel.at[pl.ds(0, nk_lo * _BLK)])
            pltpu.sync_copy(cf.at[pl.ds(ebase_w, nk_lo * _BLK)],
                            scf.at[pl.ds(0, nk_lo * _BLK)])

        def start_gather(kb, b):
            pltpu.async_copy(
                table.at[sidx.at[pl.ds(kb * _BLK, _BLK)]], rows[b], gsem[b])

        def compute_block(kb, b):
            def group_body(g, c2):
                gbase = g * _L
                el_vec = sel[pl.ds(kb * _BLK + gbase, _L)]
                cf_vec = scf[pl.ds(kb * _BLK + gbase, _L)]
                for e_l in range(_L):
                    ei = jnp.full((_L,), e_l, jnp.int32)
                    nel = -el_vec.at[ei].get(mode="promise_in_bounds")
                    cf_b = cf_vec.at[ei].get(mode="promise_in_bounds")
                    e = gbase + e_l
                    for j in range(njc):
                        s = jnp.exp(nel * inv_vecs[j]) * (pre_vecs[j] * cf_b)
                        r = rows[b][e, pl.ds(j * _L, _L)]
                        obuf[b][pl.ds(e * _D + j * _L, _L)] = r * s
                return c2

            lax.fori_loop(0, _BLK // _L, group_body, 0)
            pltpu.async_copy(
                obuf[b],
                out.at[pl.ds((ebase_w + kb * _BLK) * _D, _BLK * _D)], osem[b])

        start_gather(0, 0)

        def outer(k2, carry):
            for par in range(2):
                k = k2 * 2 + par
                b = par

                @pl.when(k + 1 < nk)
                def _(k=k, b=b):
                    start_gather(k + 1, 1 - b)

                @pl.when(k < nk)
                def _(k=k, b=b):
                    pltpu.make_async_copy(
                        table.at[sidx.at[pl.ds(k * _BLK, _BLK)]], rows[b],
                        gsem[b]).wait()

                    @pl.when(k >= 2)
                    def _():
                        pltpu.make_async_copy(
                            obuf[b], out.at[pl.ds(0, _BLK * _D)],
                            osem[b]).wait()

                    compute_block(k, b)
            return carry

        lax.fori_loop(0, (nk_hi + 1) // 2, outer, 0)

        for b in range(2):
            pltpu.make_async_copy(
                obuf[b], out.at[pl.ds(0, _BLK * _D)], osem[b]).wait()

    return sc_kernel


def kernel(node_feat, edge_lengths, radial_cutoff_fn, edge_index, prefactor, invr0):
    N = node_feat.shape[0]
    E = edge_index.shape[1]
    table = node_feat.reshape(N, _D)
    inv_flat = _expand_params(invr0)
    pre_flat = _expand_params(prefactor)
    out = _make_sc_call(E, N)(table, edge_index[0], edge_lengths,
                              radial_cutoff_fn, inv_flat, pre_flat)
    return out.reshape(E, _R, _A, _C)


# speedup vs baseline: 5.8348x; 5.8348x over previous
import functools

import jax
import jax.numpy as jnp
from jax import lax
from jax.experimental import pallas as pl
from jax.experimental.pallas import tpu as pltpu
from jax.experimental.pallas import tpu_sc as plsc

_GROUPS = ((0, 1), (1, 4), (4, 10))
_R, _A, _C = 4, 10, 8
_D = _R * _A * _C
_L = 16
_NW = 32
_BLK = 128
_NBUF = 3


def _expand_params(p):
    parts = [jnp.broadcast_to(p[g][:, None, :], (_R, e - s, _C))
             for g, (s, e) in enumerate(_GROUPS)]
    return jnp.concatenate(parts, axis=1).reshape(_D)


def _make_sc_call(E, N):
    nblk = E // _BLK
    nk_max = -(-nblk // _NW)
    nk_pad = -(-nk_max // _NBUF) * _NBUF
    mesh = plsc.VectorSubcoreMesh(core_axis_name="c", subcore_axis_name="s")
    njc = _D // _L

    @functools.partial(
        pl.kernel,
        mesh=mesh,
        compiler_params=pltpu.CompilerParams(use_tc_tiling_on_sc=False),
        out_type=jax.ShapeDtypeStruct((E, _D), jnp.float32),
        scratch_types=(
            [pltpu.VMEM((1, _BLK), jnp.int32) for _ in range(_NBUF)]
            + [pltpu.VMEM((2, _BLK), jnp.float32) for _ in range(_NBUF)]
            + [pltpu.VMEM((_BLK, _D), jnp.float32) for _ in range(_NBUF)]
            + [pltpu.VMEM((_D,), jnp.float32), pltpu.VMEM((_D,), jnp.float32)]
            + [pltpu.SemaphoreType.DMA for _ in range(2 * _NBUF)]
        ),
    )
    def sc_kernel(table, src, elcf, inv, pre, out, *refs):
        pidx = refs[0:_NBUF]
        pec = refs[_NBUF:2 * _NBUF]
        rows = refs[2 * _NBUF:3 * _NBUF]
        inv_v, pre_v = refs[3 * _NBUF], refs[3 * _NBUF + 1]
        gsem = refs[3 * _NBUF + 2:3 * _NBUF + 2 + _NBUF]
        osem = refs[3 * _NBUF + 2 + _NBUF:]

        wid = lax.axis_index("s") * 2 + lax.axis_index("c")

        pltpu.sync_copy(inv, inv_v)
        pltpu.sync_copy(pre, pre_v)
        inv_vecs = [inv_v[pl.ds(j * _L, _L)] for j in range(njc)]
        pre_vecs = [pre_v[pl.ds(j * _L, _L)] for j in range(njc)]

        def bid_of(k):
            return wid + k * _NW

        def load_block(k, b):
            base = bid_of(k) * _BLK
            pltpu.sync_copy(elcf.at[:, pl.ds(base, _BLK)], pec[b])
            pltpu.sync_copy(src.at[:, pl.ds(base, _BLK)], pidx[b])
            pltpu.async_copy(table.at[pidx[b].at[0]], rows[b], gsem[b])

        def compute_block(k, b):
            base = bid_of(k) * _BLK

            def group_body(g, c2):
                gbase = g * _L
                el_vec = pec[b][0, pl.ds(gbase, _L)]
                cf_vec = pec[b][1, pl.ds(gbase, _L)]
                for e_l in range(_L):
                    ei = jnp.full((_L,), e_l, jnp.int32)
                    nel = -el_vec.at[ei].get(mode="promise_in_bounds")
                    cf_b = cf_vec.at[ei].get(mode="promise_in_bounds")
                    e = gbase + e_l
                    for j in range(njc):
                        sl = pl.ds(j * _L, _L)
                        s = jnp.exp(nel * inv_vecs[j]) * (pre_vecs[j] * cf_b)
                        rows[b][e, sl] = rows[b][e, sl] * s
                return c2

            lax.fori_loop(0, _BLK // _L, group_body, 0)
            pltpu.async_copy(rows[b], out.at[pl.ds(base, _BLK)], osem[b])

        load_block(0, 0)

        def outer(k3, carry):
            for joff in range(_NBUF):
                k = k3 * _NBUF + joff
                b = joff
                bn = (joff + 1) % _NBUF

                @pl.when(bid_of(k + 1) < nblk)
                def _(k=k, b=b, bn=bn):
                    pltpu.sync_copy(
                        elcf.at[:, pl.ds(bid_of(k + 1) * _BLK, _BLK)], pec[bn])
                    pltpu.sync_copy(
                        src.at[:, pl.ds(bid_of(k + 1) * _BLK, _BLK)], pidx[bn])

                    @pl.when(k >= 2)
                    def _():
                        pltpu.make_async_copy(
                            rows[bn], out.at[pl.ds(bid_of(k + 1) * _BLK, _BLK)],
                            osem[bn]).wait()

                    pltpu.async_copy(table.at[pidx[bn].at[0]], rows[bn], gsem[bn])

                @pl.when(bid_of(k) < nblk)
                def _(k=k, b=b):
                    pltpu.make_async_copy(
                        table.at[pidx[b].at[0]], rows[b], gsem[b]).wait()
                    compute_block(k, b)
            return carry

        lax.fori_loop(0, nk_pad // _NBUF, outer, 0)

        for c in range(_NBUF):
            @pl.when(bid_of(c) < nblk)
            def _(c=c):
                pltpu.make_async_copy(
                    rows[c], out.at[pl.ds(0, _BLK)], osem[c]).wait()

    return sc_kernel


def kernel(node_feat, edge_lengths, radial_cutoff_fn, edge_index, prefactor, invr0):
    N = node_feat.shape[0]
    E = edge_index.shape[1]
    table = node_feat.reshape(N, _D)
    src = edge_index[0:1]
    elcf = jnp.stack([edge_lengths, radial_cutoff_fn])
    inv_flat = _expand_params(invr0)
    pre_flat = _expand_params(prefactor)
    out = _make_sc_call(E, N)(table, src, elcf, inv_flat, pre_flat)
    return out.reshape(E, _R, _A, _C)
